# Initial kernel scaffold; baseline (speedup 1.0000x reference)
import jax
import jax.numpy as jnp
from jax.experimental import pallas as pl


def kernel(x, hyperedge_index, hyperedge_weight, Wn0, bn0, Wh0, bh0, cg0, cb0, og0, ob0, Wn1, bn1, Wh1, bh1, cg1, cb1, og1, ob1):
    raise NotImplementedError("write your pallas kernel here")


# speedup vs baseline: 6.5935x; 6.5935x over previous
import functools

import jax
import jax.numpy as jnp
from jax import lax
from jax.experimental import pallas as pl
from jax.experimental.pallas import tpu as pltpu
from jax.experimental.pallas import tpu_sc as plsc

_N = 10000
_E = 320000
_D = 128
_H = 10000

_NC = 2
_NS = 16
_NW = _NC * _NS
_EPT = _E // _NW
_B = 100
_NB = _EPT // _B
_RPT = 624
_TAIL = _N - _NS * _RPT

_mesh = plsc.VectorSubcoreMesh(core_axis_name="c", subcore_axis_name="s")


def _striped_copy(t, src, dst):
  pltpu.sync_copy(src.at[pl.ds(t * _RPT, _RPT)],
                  dst.at[pl.ds(t * _RPT, _RPT)])

  @pl.when(t == _NS - 1)
  def _():
    pltpu.sync_copy(src.at[pl.ds(_NS * _RPT, _TAIL)],
                    dst.at[pl.ds(_NS * _RPT, _TAIL)])


def _seg_scatter(table, gidx, sidx, zeros):
  @functools.partial(
      pl.kernel,
      out_type=jax.ShapeDtypeStruct((_NC, _N, _D), jnp.float32),
      mesh=_mesh,
      scratch_types=[
          pltpu.VMEM((_NB, _B), jnp.int32),
          pltpu.VMEM((_NB, _B), jnp.int32),
          pltpu.VMEM((_B, _D), jnp.float32),
          pltpu.VMEM_SHARED((_N, _D), jnp.float32),
          pltpu.SemaphoreType.DMA,
      ],
  )
  def k(table_h, gidx_h, sidx_h, zeros_h, out_h, gv, sv, rows, acc, sem):
    c = lax.axis_index("c")
    t = lax.axis_index("s")
    wid = t * _NC + c
    pltpu.sync_copy(gidx_h.at[wid], gv)
    pltpu.sync_copy(sidx_h.at[wid], sv)
    _striped_copy(t, zeros_h, acc)
    plsc.subcore_barrier()

    def body(g, carry):
      pltpu.async_copy(table_h.at[gv.at[g]], rows, sem).wait()
      pltpu.sync_copy(rows, acc.at[sv.at[g]], add=True)
      return carry

    lax.fori_loop(0, _NB, body, 0)
    plsc.subcore_barrier()
    _striped_copy(t, acc, out_h.at[c])

  return k(table, gidx, sidx, zeros)


def _seg_counts(hidx, nidx, onesrow, zeros16):
  @functools.partial(
      pl.kernel,
      out_type=(jax.ShapeDtypeStruct((_NC, _H, 16), jnp.float32),
                jax.ShapeDtypeStruct((_NC, _N, 16), jnp.float32)),
      mesh=_mesh,
      scratch_types=[
          pltpu.VMEM((_NB, _B), jnp.int32),
          pltpu.VMEM((_NB, _B), jnp.int32),
          pltpu.VMEM((_B, 16), jnp.float32),
          pltpu.VMEM_SHARED((_H, 16), jnp.float32),
          pltpu.VMEM_SHARED((_N, 16), jnp.float32),
      ],
  )
  def k(hidx_h, nidx_h, ones_h, z16_h, outh_h, outn_h, hv, nv, ones, acch,
        accn):
    c = lax.axis_index("c")
    t = lax.axis_index("s")
    wid = t * _NC + c
    pltpu.sync_copy(hidx_h.at[wid], hv)
    pltpu.sync_copy(nidx_h.at[wid], nv)
    pltpu.sync_copy(ones_h, ones)
    _striped_copy(t, z16_h, acch)
    _striped_copy(t, z16_h, accn)
    plsc.subcore_barrier()

    def body(g, carry):
      pltpu.sync_copy(ones, acch.at[hv.at[g]], add=True)
      pltpu.sync_copy(ones, accn.at[nv.at[g]], add=True)
      return carry

    lax.fori_loop(0, _NB, body, 0)
    plsc.subcore_barrier()
    _striped_copy(t, acch, outh_h.at[c])
    _striped_copy(t, accn, outn_h.at[c])

  return k(hidx, nidx, onesrow, zeros16)


_BLK = 1000
_G = _N // _BLK

_row_spec = pl.BlockSpec((_BLK, _D), lambda i: (i, 0))
_cnt_spec = pl.BlockSpec((_BLK, 16), lambda i: (i, 0))
_w_spec = pl.BlockSpec((_D, _D), lambda i: (0, 0))
_vec_spec = pl.BlockSpec((1, _D), lambda i: (0, 0))
_out_sds = jax.ShapeDtypeStruct((_N, _D), jnp.float32)


def _ln(v, g, b):
  m = jnp.mean(v, axis=-1, keepdims=True)
  var = jnp.mean(v * v, axis=-1, keepdims=True) - m * m
  return (v - m) * jax.lax.rsqrt(var + 1e-5) * g + b


def _dense_in(x, wt, b):
  def body(x_ref, w_ref, b_ref, o_ref):
    o_ref[...] = jnp.dot(x_ref[...], w_ref[...],
                         preferred_element_type=jnp.float32) + b_ref[...]

  return pl.pallas_call(
      body,
      grid=(_G,),
      in_specs=[_row_spec, _w_spec, _vec_spec],
      out_specs=_row_spec,
      out_shape=_out_sds,
  )(x, wt, b.reshape(1, _D))


def _dense_he(agg, cnth, wt, b, w16):
  def body(a0_ref, a1_ref, c0_ref, c1_ref, w_ref, b_ref, hw_ref, o_ref):
    cnt = c0_ref[...][:, 0:1] + c1_ref[...][:, 0:1]
    he = (a0_ref[...] + a1_ref[...]) / (cnt + 1e-8)
    he = jnp.dot(he, w_ref[...], preferred_element_type=jnp.float32)
    he = he + b_ref[...]
    o_ref[...] = he * hw_ref[...][:, 0:1]

  return pl.pallas_call(
      body,
      grid=(_G,),
      in_specs=[_row_spec, _row_spec, _cnt_spec, _cnt_spec, _w_spec,
                _vec_spec, _cnt_spec],
      out_specs=_row_spec,
      out_shape=_out_sds,
  )(agg[0], agg[1], cnth[0], cnth[1], wt, b.reshape(1, _D), w16)


def _dense_out(sums, cntn, xt, res, cg, cb, og, ob):
  add_res = res is not None

  def body(*refs):
    (s0_ref, s1_ref, c0_ref, c1_ref, xt_ref), rest = refs[:5], refs[5:]
    if add_res:
      res_ref, rest = rest[0], rest[1:]
    cg_ref, cb_ref, og_ref, ob_ref, o_ref = rest
    cnt = jnp.maximum(c0_ref[...][:, 0:1] + c1_ref[...][:, 0:1], 1.0)
    t = (s0_ref[...] + s1_ref[...]) / cnt + xt_ref[...]
    t = _ln(t, cg_ref[...], cb_ref[...])
    t = jnp.where(t > 0, t, 0.2 * t)
    t = _ln(t, og_ref[...], ob_ref[...])
    if add_res:
      t = t + res_ref[...]
    o_ref[...] = t

  in_specs = [_row_spec, _row_spec, _cnt_spec, _cnt_spec, _row_spec]
  args = [sums[0], sums[1], cntn[0], cntn[1], xt]
  if add_res:
    in_specs.append(_row_spec)
    args.append(res)
  in_specs += [_vec_spec] * 4
  args += [cg.reshape(1, _D), cb.reshape(1, _D), og.reshape(1, _D),
           ob.reshape(1, _D)]

  return pl.pallas_call(
      body,
      grid=(_G,),
      in_specs=in_specs,
      out_specs=_row_spec,
      out_shape=_out_sds,
  )(*args)


def kernel(x, hyperedge_index, hyperedge_weight, Wn0, bn0, Wh0, bh0, cg0,
           cb0, og0, ob0, Wn1, bn1, Wh1, bh1, cg1, cb1, og1, ob1):
  nidx = hyperedge_index[0].astype(jnp.int32).reshape(_NW, _NB, _B)
  hidx = hyperedge_index[1].astype(jnp.int32).reshape(_NW, _NB, _B)
  zeros_d = jnp.zeros((_N, _D), jnp.float32)
  zeros_16 = jnp.zeros((_N, 16), jnp.float32)
  onesrow = jnp.pad(jnp.ones((_B, 1), jnp.float32), ((0, 0), (0, 15)))
  w16 = jnp.pad(hyperedge_weight.reshape(_H, 1), ((0, 0), (0, 15)))

  cnth, cntn = _seg_counts(hidx, nidx, onesrow, zeros_16)

  xt0 = _dense_in(x, Wn0.T, bn0)
  agg0 = _seg_scatter(xt0, nidx, hidx, zeros_d)
  he0 = _dense_he(agg0, cnth, Wh0.T, bh0, w16)
  sm0 = _seg_scatter(he0, hidx, nidx, zeros_d)
  x1 = _dense_out(sm0, cntn, xt0, None, cg0, cb0, og0, ob0)

  xt1 = _dense_in(x1, Wn1.T, bn1)
  agg1 = _seg_scatter(xt1, nidx, hidx, zeros_d)
  he1 = _dense_he(agg1, cnth, Wh1.T, bh1, w16)
  sm1 = _seg_scatter(he1, hidx, nidx, zeros_d)
  return _dense_out(sm1, cntn, xt1, x1, cg1, cb1, og1, ob1)
